# baseline (speedup 1.0000x reference)
import functools

import jax
import jax.numpy as jnp
from jax import lax
from jax.experimental import pallas as pl
from jax.experimental.pallas import tpu as pltpu
from jax.experimental.pallas import tpu_sc as plsc

_LANES = 16


@functools.cache
def _pos_add_call(B, L, D):
  info = plsc.get_sparse_core_info()
  NC, NS = info.num_cores, info.num_subcores
  NW = NC * NS
  LW = L // NW
  C = min(LW, 32)
  NCHUNK = LW // C
  CW = C * D
  NV = CW // _LANES
  mesh = plsc.VectorSubcoreMesh(core_axis_name="c", subcore_axis_name="s")

  @functools.partial(
      pl.kernel,
      out_type=jax.ShapeDtypeStruct((B * L * D,), jnp.float32),
      mesh=mesh,
      scratch_types=[
          pltpu.VMEM((CW,), jnp.float32),
          pltpu.VMEM((CW,), jnp.float32),
      ],
  )
  def k(x_hbm, pos_hbm, out_hbm, pos_v, x_v):
    wid = lax.axis_index("s") * NC + lax.axis_index("c")
    lbase = wid * LW
    for lc in range(NCHUNK):
      pos_off = pl.multiple_of((lbase + lc * C) * D, D)
      pltpu.sync_copy(pos_hbm.at[pl.ds(pos_off, CW)], pos_v)
      for b in range(B):
        x_off = pl.multiple_of((b * L + lbase + lc * C) * D, D)
        pltpu.sync_copy(x_hbm.at[pl.ds(x_off, CW)], x_v)

        @plsc.parallel_loop(0, NV, unroll=8)
        def _(i):
          sl = pl.ds(i * _LANES, _LANES)
          x_v[sl] = x_v[sl] + pos_v[sl]

        pltpu.sync_copy(x_v, out_hbm.at[pl.ds(x_off, CW)])

  return k


def kernel(x, pos_table):
  B, L, D = x.shape
  x_flat = x.reshape(-1)
  pos_flat = pos_table[:L].reshape(-1)
  out = _pos_add_call(B, L, D)(x_flat, pos_flat)
  return out.reshape(B, L, D)


# speedup vs baseline: 1.1588x; 1.1588x over previous
import functools

import jax
import jax.numpy as jnp
from jax import lax
from jax.experimental import pallas as pl
from jax.experimental.pallas import tpu as pltpu
from jax.experimental.pallas import tpu_sc as plsc

_LANES = 16


@functools.cache
def _pos_add_call(B, L, D):
  info = plsc.get_sparse_core_info()
  NC, NS = info.num_cores, info.num_subcores
  NW = NC * NS
  LW = L // NW
  C = min(LW, 16)
  NCHUNK = LW // C
  CW = C * D
  NV = CW // _LANES
  NITEMS = NCHUNK * B
  mesh = plsc.VectorSubcoreMesh(core_axis_name="c", subcore_axis_name="s")

  @functools.partial(
      pl.kernel,
      out_type=jax.ShapeDtypeStruct((B * L * D,), jnp.float32),
      mesh=mesh,
      scratch_types=[
          [pltpu.VMEM((CW,), jnp.float32)] * 2,
          [pltpu.VMEM((CW,), jnp.float32)] * 2,
          [pltpu.VMEM((CW,), jnp.float32)] * 2,
          [pltpu.SemaphoreType.DMA] * 2,
          [pltpu.SemaphoreType.DMA] * 2,
          [pltpu.SemaphoreType.DMA] * 2,
      ],
  )
  def k(x_hbm, pos_hbm, out_hbm, pos_v, in_v, out_v, sp, sl, ss):
    wid = lax.axis_index("s") * NC + lax.axis_index("c")
    lbase = wid * LW

    def x_off(item):
      lc, b = divmod(item, B)
      return pl.multiple_of((b * L + lbase + lc * C) * D, D)

    def pos_load(lc):
      off = pl.multiple_of((lbase + lc * C) * D, D)
      return pltpu.async_copy(pos_hbm.at[pl.ds(off, CW)], pos_v[lc % 2],
                              sp[lc % 2])

    def x_load(item):
      return pltpu.async_copy(x_hbm.at[pl.ds(x_off(item), CW)],
                              in_v[item % 2], sl[item % 2])

    pos_d = [None] * NCHUNK
    loads = [None] * NITEMS
    stores = [None] * NITEMS
    pos_d[0] = pos_load(0)
    loads[0] = x_load(0)
    if NITEMS > 1:
      loads[1] = x_load(1)

    for item in range(NITEMS):
      lc, b = divmod(item, B)
      if b == 0:
        pos_d[lc].wait()
        if lc + 1 < NCHUNK:
          pos_d[lc + 1] = pos_load(lc + 1)
      loads[item].wait()
      if item >= 2:
        stores[item - 2].wait()
      src, dst, pos = in_v[item % 2], out_v[item % 2], pos_v[lc % 2]

      @plsc.parallel_loop(0, NV, unroll=8)
      def _(i, _src=src, _dst=dst, _pos=pos):
        s = pl.ds(i * _LANES, _LANES)
        _dst[s] = _src[s] + _pos[s]

      stores[item] = pltpu.async_copy(out_v[item % 2],
                                      out_hbm.at[pl.ds(x_off(item), CW)],
                                      ss[item % 2])
      if item + 2 < NITEMS:
        loads[item + 2] = x_load(item + 2)

    for d in stores[-2:]:
      if d is not None:
        d.wait()

  return k


def kernel(x, pos_table):
  B, L, D = x.shape
  x_flat = x.reshape(-1)
  pos_flat = pos_table[:L].reshape(-1)
  out = _pos_add_call(B, L, D)(x_flat, pos_flat)
  return out.reshape(B, L, D)


# speedup vs baseline: 2.8474x; 2.4573x over previous
import functools

import jax
import jax.numpy as jnp
from jax import lax
from jax.experimental import pallas as pl
from jax.experimental.pallas import tpu as pltpu
from jax.experimental.pallas import tpu_sc as plsc

_LANES = 16


@functools.cache
def _pos_add_call(B, L, D):
  info = plsc.get_sparse_core_info()
  NC, NS = info.num_cores, info.num_subcores
  NW = NC * NS
  LW = L // NW
  C = min(LW, 16)
  NCHUNK = LW // C
  NV = C * D // _LANES
  NCOL = D // _LANES
  NITEMS = NCHUNK * B
  mesh = plsc.VectorSubcoreMesh(core_axis_name="c", subcore_axis_name="s")

  @functools.partial(
      pl.kernel,
      out_type=jax.ShapeDtypeStruct((B * L, D), jnp.float32),
      mesh=mesh,
      scratch_types=[
          [pltpu.VMEM((C, D), jnp.float32)] * 2,
          [pltpu.VMEM((C, D), jnp.float32)] * 2,
          [pltpu.VMEM((C, D), jnp.float32)] * 2,
          [pltpu.SemaphoreType.DMA] * 2,
          [pltpu.SemaphoreType.DMA] * 2,
          [pltpu.SemaphoreType.DMA] * 2,
      ],
  )
  def k(x_hbm, pos_hbm, out_hbm, pos_v, in_v, out_v, sp, sl, ss):
    wid = lax.axis_index("s") * NC + lax.axis_index("c")
    lbase = wid * LW

    def x_row(item):
      lc, b = divmod(item, B)
      return pl.multiple_of(b * L + lbase + lc * C, C)

    def pos_load(lc):
      row = pl.multiple_of(lbase + lc * C, C)
      return pltpu.async_copy(pos_hbm.at[pl.ds(row, C)], pos_v[lc % 2],
                              sp[lc % 2])

    def x_load(item):
      return pltpu.async_copy(x_hbm.at[pl.ds(x_row(item), C)],
                              in_v[item % 2], sl[item % 2])

    pos_d = [None] * NCHUNK
    loads = [None] * NITEMS
    stores = [None] * NITEMS
    pos_d[0] = pos_load(0)
    loads[0] = x_load(0)
    if NITEMS > 1:
      loads[1] = x_load(1)

    for item in range(NITEMS):
      lc, b = divmod(item, B)
      if b == 0:
        pos_d[lc].wait()
        if lc + 1 < NCHUNK:
          pos_d[lc + 1] = pos_load(lc + 1)
      loads[item].wait()
      if item >= 2:
        stores[item - 2].wait()
      src, dst, pos = in_v[item % 2], out_v[item % 2], pos_v[lc % 2]

      @plsc.parallel_loop(0, NV, unroll=8)
      def _(i, _src=src, _dst=dst, _pos=pos):
        r = i // NCOL
        s = pl.ds((i % NCOL) * _LANES, _LANES)
        _dst[r, s] = _src[r, s] + _pos[r, s]

      stores[item] = pltpu.async_copy(out_v[item % 2],
                                      out_hbm.at[pl.ds(x_row(item), C)],
                                      ss[item % 2])
      if item + 2 < NITEMS:
        loads[item + 2] = x_load(item + 2)

    for d in stores[-2:]:
      if d is not None:
        d.wait()

  return k


def kernel(x, pos_table):
  B, L, D = x.shape
  out = _pos_add_call(B, L, D)(x.reshape(B * L, D), pos_table[:L])
  return out.reshape(B, L, D)
